# baseline (speedup 1.0000x reference)
import functools

import jax
import jax.numpy as jnp
import numpy as np
from jax.experimental import pallas as pl
from jax.experimental.pallas import tpu as pltpu

_TEMPERATURE = 0.02
_K = 10
_PRE = 5
_NEG = float(np.finfo(np.float32).min)


def _kth_largest_rows(X, k):
    blk, B = X.shape
    nch = B // 128
    cm = X.reshape(blk, nch, 128)
    cands = []
    t4 = None
    for _ in range(_PRE):
        t4 = jnp.max(cm, axis=2, keepdims=True)
        cm = jnp.where(cm >= t4, _NEG, cm)
        cands.append(t4.reshape(blk, nch))
    rowmax = jnp.max(cands[0], axis=1, keepdims=True)
    cand = jnp.concatenate(cands, axis=1)
    tm = cand
    tc = None
    for _ in range(k):
        tc = jnp.max(tm, axis=1, keepdims=True)
        tm = jnp.where(tm >= tc, _NEG, tm)
    cnt = jnp.sum((X >= tc).astype(jnp.float32), axis=1, keepdims=True)
    bad = jnp.any(cnt != float(k))

    def full():
        xm = X
        t = None
        for _ in range(k):
            t = jnp.max(xm, axis=1, keepdims=True)
            xm = jnp.where(xm >= t, _NEG, xm)
        return t

    t_final = jax.lax.cond(bad, full, lambda: tc)
    return t_final, rowmax


def _kth_largest_cols(Y, k):
    B, blk = Y.shape
    nch = B // 128
    cm = Y.reshape(nch, 128, blk)
    cands = []
    t4 = None
    for _ in range(_PRE):
        t4 = jnp.max(cm, axis=1, keepdims=True)
        cm = jnp.where(cm >= t4, _NEG, cm)
        cands.append(t4.reshape(nch, blk))
    cand = jnp.concatenate(cands, axis=0)
    tm = cand
    tc = None
    for _ in range(k):
        tc = jnp.max(tm, axis=0, keepdims=True)
        tm = jnp.where(tm >= tc, _NEG, tm)
    cnt = jnp.sum((Y >= tc).astype(jnp.float32), axis=0, keepdims=True)
    bad = jnp.any(cnt != float(k))

    def full():
        ym = Y
        t = None
        for _ in range(k):
            t = jnp.max(ym, axis=0, keepdims=True)
            ym = jnp.where(ym >= t, _NEG, ym)
        return t

    return jax.lax.cond(bad, full, lambda: tc)


def _fused_body(row_ref, col_ref, unc_ref, ent_ref, *, k: int, max_ent: float):
    X = row_ref[...]
    Y = col_ref[...]
    blk = X.shape[0]

    tr, rowmax = _kth_largest_rows(X, k)
    tc = _kth_largest_cols(Y, k)

    inv_t = 1.0 / _TEMPERATURE
    m = rowmax * inv_t
    sm = X * inv_t - m
    e = jnp.exp(sm)
    Z = jnp.sum(e, axis=1, keepdims=True)
    S1 = jnp.sum(sm * e, axis=1, keepdims=True)
    ent = (jnp.log(Z) - S1 / Z)[:, 0] * (1.0 / max_ent)

    R = (X >= tr).astype(jnp.float32)
    C = (Y >= tc).astype(jnp.float32)
    P = jax.lax.dot(R, C, preferred_element_type=jnp.float32)
    ii = jax.lax.broadcasted_iota(jnp.int32, (blk, blk), 0)
    jj = jax.lax.broadcasted_iota(jnp.int32, (blk, blk), 1)
    matches = jnp.sum(jnp.where(ii == jj, P, 0.0), axis=1)

    ra = matches * (1.0 / k)
    unc_ref[...] = (1.0 - ra) * 0.5 + ent * 0.5
    ent_ref[...] = ent


def kernel(sim_matrix, pids):
    del pids
    B = sim_matrix.shape[0]
    blk = 256
    k = min(_K, B)
    max_ent = float(np.log(B + 1e-10))
    grid = B // blk
    unc, ent = pl.pallas_call(
        functools.partial(_fused_body, k=k, max_ent=max_ent),
        grid=(grid,),
        in_specs=[
            pl.BlockSpec((blk, B), lambda i: (i, 0)),
            pl.BlockSpec((B, blk), lambda i: (0, i)),
        ],
        out_specs=[
            pl.BlockSpec((blk,), lambda i: (i,)),
            pl.BlockSpec((blk,), lambda i: (i,)),
        ],
        out_shape=[
            jax.ShapeDtypeStruct((B,), jnp.float32),
            jax.ShapeDtypeStruct((B,), jnp.float32),
        ],
    )(sim_matrix, sim_matrix)
    return (unc, ent)


# speedup vs baseline: 1.4265x; 1.4265x over previous
import functools

import jax
import jax.numpy as jnp
import numpy as np
from jax.experimental import pallas as pl
from jax.experimental.pallas import tpu as pltpu

_TEMPERATURE = 0.02
_K = 10
_NEG = float(np.finfo(np.float32).min)


def _fused_body(row_ref, col_ref, unc_ref, ent_ref, *, k: int, max_ent: float):
    X = row_ref[...]
    Y = col_ref[...]
    blk = X.shape[0]

    xm = X
    tr = None
    rowmax = None
    for it in range(k):
        tr = jnp.max(xm, axis=1, keepdims=True)
        if it == 0:
            rowmax = tr
        xm = jnp.where(xm >= tr, _NEG, xm)

    ym = Y
    tc = None
    for _ in range(k):
        tc = jnp.max(ym, axis=0, keepdims=True)
        ym = jnp.where(ym >= tc, _NEG, ym)

    inv_t = 1.0 / _TEMPERATURE
    sm = (X - rowmax) * inv_t
    e = jnp.exp(sm)
    Z = jnp.sum(e, axis=1, keepdims=True)
    S1 = jnp.sum(sm * e, axis=1, keepdims=True)
    ent = (jnp.log(Z) - S1 / Z)[:, 0] * (1.0 / max_ent)

    R = (X >= tr).astype(jnp.float32)
    C = (Y >= tc).astype(jnp.float32)
    P = jax.lax.dot(R, C, preferred_element_type=jnp.float32)
    ii = jax.lax.broadcasted_iota(jnp.int32, (blk, blk), 0)
    jj = jax.lax.broadcasted_iota(jnp.int32, (blk, blk), 1)
    matches = jnp.sum(jnp.where(ii == jj, P, 0.0), axis=1)

    ra = matches * (1.0 / k)
    unc_ref[...] = (1.0 - ra) * 0.5 + ent * 0.5
    ent_ref[...] = ent


def kernel(sim_matrix, pids):
    del pids
    B = sim_matrix.shape[0]
    blk = 512
    k = min(_K, B)
    max_ent = float(np.log(B + 1e-10))
    grid = B // blk
    unc, ent = pl.pallas_call(
        functools.partial(_fused_body, k=k, max_ent=max_ent),
        grid=(grid,),
        in_specs=[
            pl.BlockSpec((blk, B), lambda i: (i, 0)),
            pl.BlockSpec((B, blk), lambda i: (0, i)),
        ],
        out_specs=[
            pl.BlockSpec((blk,), lambda i: (i,)),
            pl.BlockSpec((blk,), lambda i: (i,)),
        ],
        out_shape=[
            jax.ShapeDtypeStruct((B,), jnp.float32),
            jax.ShapeDtypeStruct((B,), jnp.float32),
        ],
    )(sim_matrix, sim_matrix)
    return (unc, ent)
